# Initial kernel scaffold; baseline (speedup 1.0000x reference)
import jax
import jax.numpy as jnp
from jax.experimental import pallas as pl


def kernel(positions, edge_index, bond_types, batch):
    raise NotImplementedError("write your pallas kernel here")


# speedup vs baseline: 27.4414x; 27.4414x over previous
import functools

import jax
import jax.numpy as jnp
from jax import lax
from jax.experimental import pallas as pl
from jax.experimental.pallas import tpu as pltpu
from jax.experimental.pallas import tpu_sc as plsc

N_EDGES = 6_400_000
N_NODES = 100_000
NC, NS, LANES = 2, 16, 16
NW = NC * NS
CHUNK = 2048
G = CHUNK // 128
NCHUNKS = N_EDGES // CHUNK
BASE_PER_W = NCHUNKS // NW
EXTRA = NCHUNKS - BASE_PER_W * NW


def _sqrt16(s):
    i = lax.bitcast_convert_type(s, jnp.int32)
    x = lax.bitcast_convert_type((i >> 1) + jnp.int32(0x1FBD1DF6),
                                 jnp.float32)
    x = 0.5 * (x + s / x)
    x = 0.5 * (x + s / x)
    return x


def _body(px, py, pz, row2d, col2d, bt, out,
          rowidx, colidx, btbuf, rx, ry, rz, cx, cy, cz, accbuf, gsem):
    cid = lax.axis_index("c")
    sid = lax.axis_index("s")
    wid = sid * NC + cid

    start = BASE_PER_W * wid + jnp.minimum(wid, EXTRA)
    count = BASE_PER_W + jnp.where(wid < EXTRA, 1, 0)

    def chunk_body(k, acc):
        c = start + k
        pltpu.sync_copy(row2d.at[pl.ds(c * G, G)], rowidx)
        pltpu.sync_copy(col2d.at[pl.ds(c * G, G)], colidx)
        pltpu.sync_copy(bt.at[pl.ds(c * CHUNK, CHUNK)], btbuf)

        def fire(j, t):
            sl = pl.ds(j * 128, 128)
            pltpu.async_copy(px.at[rowidx.at[j]], rx.at[sl], gsem)
            pltpu.async_copy(py.at[rowidx.at[j]], ry.at[sl], gsem)
            pltpu.async_copy(pz.at[rowidx.at[j]], rz.at[sl], gsem)
            pltpu.async_copy(px.at[colidx.at[j]], cx.at[sl], gsem)
            pltpu.async_copy(py.at[colidx.at[j]], cy.at[sl], gsem)
            pltpu.async_copy(pz.at[colidx.at[j]], cz.at[sl], gsem)
            return t
        lax.fori_loop(0, G, fire, 0, unroll=False)
        for buf in (rx, ry, rz, cx, cy, cz):
            pltpu.make_async_copy(px.at[pl.ds(0, CHUNK)], buf, gsem).wait()

        def group(g, a):
            sl = pl.ds(g * 16, 16)
            dx = rx[sl] - cx[sl]
            dy = ry[sl] - cy[sl]
            dz = rz[sl] - cz[sl]
            s = dx * dx + dy * dy + dz * dz
            ln = _sqrt16(s)
            t = btbuf[sl]
            e = jnp.where(t == 0, jnp.float32(1.5),
                          jnp.float32(1.4) - jnp.float32(0.1) *
                          t.astype(jnp.float32))
            d = ln - e
            return a + d * d

        csum = lax.fori_loop(0, CHUNK // 16, group,
                             jnp.zeros((16,), jnp.float32))
        return acc + csum

    acc = lax.fori_loop(0, count, chunk_body, jnp.zeros((16,), jnp.float32))
    accbuf[...] = acc
    pltpu.sync_copy(accbuf, out.at[wid])


@jax.jit
def _run(px, py, pz, row2d, col2d, bt):
    mesh = plsc.VectorSubcoreMesh(core_axis_name="c", subcore_axis_name="s",
                                  num_cores=NC, num_subcores=NS)
    f = pl.kernel(
        _body,
        out_type=jax.ShapeDtypeStruct((NW, 16), jnp.float32),
        mesh=mesh,
        scratch_types=[
            pltpu.VMEM((G, 128), jnp.int32),
            pltpu.VMEM((G, 128), jnp.int32),
            pltpu.VMEM((CHUNK,), jnp.int32),
            pltpu.VMEM((CHUNK,), jnp.float32),
            pltpu.VMEM((CHUNK,), jnp.float32),
            pltpu.VMEM((CHUNK,), jnp.float32),
            pltpu.VMEM((CHUNK,), jnp.float32),
            pltpu.VMEM((CHUNK,), jnp.float32),
            pltpu.VMEM((CHUNK,), jnp.float32),
            pltpu.VMEM((16,), jnp.float32),
            pltpu.SemaphoreType.DMA,
        ],
    )
    return f(px, py, pz, row2d, col2d, bt)


def kernel(positions, edge_index, bond_types, batch):
    px = positions[:, 0]
    py = positions[:, 1]
    pz = positions[:, 2]
    row2d = edge_index[0].reshape(N_EDGES // 128, 128)
    col2d = edge_index[1].reshape(N_EDGES // 128, 128)
    partials = _run(px, py, pz, row2d, col2d, bond_types)
    return jnp.sum(partials) / jnp.float32(N_EDGES)


# speedup vs baseline: 28.8690x; 1.0520x over previous
import functools

import jax
import jax.numpy as jnp
from jax import lax
from jax.experimental import pallas as pl
from jax.experimental.pallas import tpu as pltpu
from jax.experimental.pallas import tpu_sc as plsc

N_EDGES = 6_400_000
N_NODES = 100_000
NC, NS, LANES = 2, 16, 16
NW = NC * NS
CHUNK = 5120
NCHUNKS = N_EDGES // CHUNK
BASE_PER_W = NCHUNKS // NW
EXTRA = NCHUNKS - BASE_PER_W * NW
MAXPAIRS = (BASE_PER_W + 2) // 2


def _sqrt16(s):
    i = lax.bitcast_convert_type(s, jnp.int32)
    y = lax.bitcast_convert_type(jnp.int32(0x5F3759DF) - (i >> 1),
                                 jnp.float32)
    hs = 0.5 * s
    y = y * (1.5 - hs * y * y)
    y = y * (1.5 - hs * y * y)
    return s * y


def _body(px, py, pz, row, col, bt, out,
          rowidx0, colidx0, btbuf0, rx0, ry0, rz0, cx0, cy0, cz0,
          rowidx1, colidx1, btbuf1, rx1, ry1, rz1, cx1, cy1, cz1,
          accbuf, gsem0, gsem1):
    cid = lax.axis_index("c")
    sid = lax.axis_index("s")
    wid = sid * NC + cid

    lo = BASE_PER_W * wid + jnp.minimum(wid, EXTRA)
    count = BASE_PER_W + jnp.where(wid < EXTRA, 1, 0)

    sets = (
        (rowidx0, colidx0, btbuf0, rx0, ry0, rz0, cx0, cy0, cz0, gsem0),
        (rowidx1, colidx1, btbuf1, rx1, ry1, rz1, cx1, cy1, cz1, gsem1),
    )

    def fire(c, bufset):
        rowidx, colidx, btbuf = bufset[0], bufset[1], bufset[2]
        gsem = bufset[9]
        sl = pl.ds(c * CHUNK, CHUNK)
        pltpu.sync_copy(row.at[sl], rowidx)
        pltpu.sync_copy(col.at[sl], colidx)
        pltpu.sync_copy(bt.at[sl], btbuf)
        pltpu.async_copy(px.at[rowidx], bufset[3], gsem)
        pltpu.async_copy(py.at[rowidx], bufset[4], gsem)
        pltpu.async_copy(pz.at[rowidx], bufset[5], gsem)
        pltpu.async_copy(px.at[colidx], bufset[6], gsem)
        pltpu.async_copy(py.at[colidx], bufset[7], gsem)
        pltpu.async_copy(pz.at[colidx], bufset[8], gsem)

    def wait(bufset):
        gsem = bufset[9]
        for buf in bufset[3:9]:
            pltpu.make_async_copy(px.at[pl.ds(0, CHUNK)], buf, gsem).wait()

    def compute(bufset):
        btbuf = bufset[2]
        rx, ry, rz, cx, cy, cz = bufset[3:9]

        def group(g, a):
            sl = pl.ds(g * 16, 16)
            dx = rx[sl] - cx[sl]
            dy = ry[sl] - cy[sl]
            dz = rz[sl] - cz[sl]
            s = dx * dx + dy * dy + dz * dz
            ln = _sqrt16(s)
            t = btbuf[sl]
            e = jnp.where(t == 0, jnp.float32(1.5),
                          jnp.float32(1.4) - jnp.float32(0.1) *
                          t.astype(jnp.float32))
            d = ln - e
            return a + d * d

        csum = lax.fori_loop(0, CHUNK // 16, group,
                             jnp.zeros((16,), jnp.float32))
        accbuf[...] = accbuf[...] + csum

    accbuf[...] = jnp.zeros((16,), jnp.float32)
    fire(lo, sets[0])

    def pair(i, t):
        a = lo + 2 * i

        @pl.when(2 * i < count)
        def do_a():
            wait(sets[0])
            pl.when(2 * i + 1 < count)(lambda: fire(a + 1, sets[1]))
            compute(sets[0])

        @pl.when(2 * i + 1 < count)
        def do_b():
            wait(sets[1])
            pl.when(2 * i + 2 < count)(lambda: fire(a + 2, sets[0]))
            compute(sets[1])

        return t

    lax.fori_loop(0, MAXPAIRS, pair, 0, unroll=False)
    pltpu.sync_copy(accbuf, out.at[wid])


@jax.jit
def _run(px, py, pz, row, col, bt):
    mesh = plsc.VectorSubcoreMesh(core_axis_name="c", subcore_axis_name="s",
                                  num_cores=NC, num_subcores=NS)
    edge_bufs = [
        pltpu.VMEM((CHUNK,), jnp.int32),
        pltpu.VMEM((CHUNK,), jnp.int32),
        pltpu.VMEM((CHUNK,), jnp.int32),
        pltpu.VMEM((CHUNK,), jnp.float32),
        pltpu.VMEM((CHUNK,), jnp.float32),
        pltpu.VMEM((CHUNK,), jnp.float32),
        pltpu.VMEM((CHUNK,), jnp.float32),
        pltpu.VMEM((CHUNK,), jnp.float32),
        pltpu.VMEM((CHUNK,), jnp.float32),
    ]
    f = pl.kernel(
        _body,
        out_type=jax.ShapeDtypeStruct((NW, 16), jnp.float32),
        mesh=mesh,
        scratch_types=edge_bufs + edge_bufs + [
            pltpu.VMEM((16,), jnp.float32),
            pltpu.SemaphoreType.DMA,
            pltpu.SemaphoreType.DMA,
        ],
    )
    return f(px, py, pz, row, col, bt)


def kernel(positions, edge_index, bond_types, batch):
    px = positions[:, 0]
    py = positions[:, 1]
    pz = positions[:, 2]
    partials = _run(px, py, pz, edge_index[0], edge_index[1], bond_types)
    return jnp.sum(partials) / jnp.float32(N_EDGES)


# speedup vs baseline: 91.5888x; 3.1726x over previous
import functools

import jax
import jax.numpy as jnp
from jax import lax
from jax.experimental import pallas as pl
from jax.experimental.pallas import tpu as pltpu
from jax.experimental.pallas import tpu_sc as plsc

N_EDGES = 6_400_000
N_NODES = 100_000
NC, NS, LANES = 2, 16, 16
NW = NC * NS
CHUNK = 3200
NCHUNKS = N_EDGES // CHUNK
BASE_PER_W = NCHUNKS // NW
EXTRA = NCHUNKS - BASE_PER_W * NW
MAXPAIRS = (BASE_PER_W + 2) // 2
GROUPS = CHUNK // 16


def _sqrt16(s):
    i = lax.bitcast_convert_type(s, jnp.int32)
    y = lax.bitcast_convert_type(jnp.int32(0x5F3759DF) - (i >> 1),
                                 jnp.float32)
    hs = 0.5 * s
    y = y * (1.5 - hs * y * y)
    y = y * (1.5 - hs * y * y)
    return s * y


def _body(px, py, pz, row, col, bt, out, sout,
          table,
          ri0, ci0, sb0, bb0, ri1, ci1, sb1, bb1,
          accbuf, isem0, isem1, osem0, osem1):
    cid = lax.axis_index("c")
    sid = lax.axis_index("s")
    wid = sid * NC + cid

    lo = BASE_PER_W * wid + jnp.minimum(wid, EXTRA)
    count = BASE_PER_W + jnp.where(wid < EXTRA, 1, 0)

    sets = (
        (ri0, ci0, sb0, bb0, isem0, osem0),
        (ri1, ci1, sb1, bb1, isem1, osem1),
    )

    def fire(c, bufset, phase):
        ri, ci, sb, bb, isem, osem = bufset
        sl = pl.ds(c * CHUNK, CHUNK)
        pltpu.async_copy(row.at[sl], ri, isem)
        pltpu.async_copy(col.at[sl], ci, isem)
        if phase >= 1:
            pltpu.async_copy(sout.at[sl], sb, isem)
        if phase == 2:
            pltpu.async_copy(bt.at[sl], bb, isem)

    def wait_in(bufset, phase):
        ri, ci, sb, bb, isem, osem = bufset
        pltpu.make_async_copy(row.at[pl.ds(0, CHUNK)], ri, isem).wait()
        pltpu.make_async_copy(row.at[pl.ds(0, CHUNK)], ci, isem).wait()
        if phase >= 1:
            pltpu.make_async_copy(sout.at[pl.ds(0, CHUNK)], sb, isem).wait()
        if phase == 2:
            pltpu.make_async_copy(bt.at[pl.ds(0, CHUNK)], bb, isem).wait()

    def drain_out(bufset):
        sb, osem = bufset[2], bufset[5]
        pltpu.make_async_copy(sout.at[pl.ds(0, CHUNK)], sb, osem).wait()

    def compute(c, bufset, phase):
        ri, ci, sb, bb, isem, osem = bufset

        if phase == 2:
            def group(g, a):
                sl = pl.ds(g * 16, 16)
                rr = plsc.load_gather(table, [ri[sl]])
                cc = plsc.load_gather(table, [ci[sl]])
                d = rr - cc
                s = sb[sl] + d * d
                ln = _sqrt16(s)
                t = bb[sl]
                e = jnp.where(t == 0, jnp.float32(1.5),
                              jnp.float32(1.4) - jnp.float32(0.1) *
                              t.astype(jnp.float32))
                dd = ln - e
                return a + dd * dd
            csum = lax.fori_loop(0, GROUPS, group,
                                 jnp.zeros((16,), jnp.float32), unroll=2)
            accbuf[...] = accbuf[...] + csum
        else:
            def group(g, t):
                sl = pl.ds(g * 16, 16)
                rr = plsc.load_gather(table, [ri[sl]])
                cc = plsc.load_gather(table, [ci[sl]])
                d = rr - cc
                if phase == 0:
                    sb[sl] = d * d
                else:
                    sb[sl] = sb[sl] + d * d
                return t
            lax.fori_loop(0, GROUPS, group, 0, unroll=2)
            pltpu.async_copy(sb, sout.at[pl.ds(c * CHUNK, CHUNK)], osem)

    def run_phase(coord, phase):
        pltpu.sync_copy(coord, table)
        fire(lo, sets[0], phase)

        def pair(i, t):
            a = lo + 2 * i

            @pl.when(2 * i < count)
            def do_a():
                wait_in(sets[0], phase)

                @pl.when(2 * i + 1 < count)
                def fire_b():
                    if phase != 2:
                        pl.when(i > 0)(lambda: drain_out(sets[1]))
                    fire(a + 1, sets[1], phase)

                compute(a, sets[0], phase)

            @pl.when(2 * i + 1 < count)
            def do_b():
                wait_in(sets[1], phase)

                @pl.when(2 * i + 2 < count)
                def fire_a():
                    if phase != 2:
                        drain_out(sets[0])
                    fire(a + 2, sets[0], phase)

                compute(a + 1, sets[1], phase)

            return t

        lax.fori_loop(0, MAXPAIRS, pair, 0, unroll=False)
        if phase != 2:
            drain_out(sets[0])
            drain_out(sets[1])

    accbuf[...] = jnp.zeros((16,), jnp.float32)
    run_phase(px, 0)
    run_phase(py, 1)
    run_phase(pz, 2)
    pltpu.sync_copy(accbuf, out.at[wid])


@jax.jit
def _run(px, py, pz, row, col, bt):
    mesh = plsc.VectorSubcoreMesh(core_axis_name="c", subcore_axis_name="s",
                                  num_cores=NC, num_subcores=NS)
    edge_bufs = [
        pltpu.VMEM((CHUNK,), jnp.int32),
        pltpu.VMEM((CHUNK,), jnp.int32),
        pltpu.VMEM((CHUNK,), jnp.float32),
        pltpu.VMEM((CHUNK,), jnp.int32),
    ]
    f = pl.kernel(
        _body,
        out_type=(jax.ShapeDtypeStruct((NW, 16), jnp.float32),
                  jax.ShapeDtypeStruct((N_EDGES,), jnp.float32)),
        mesh=mesh,
        compiler_params=pltpu.CompilerParams(needs_layout_passes=False),
        scratch_types=[pltpu.VMEM((N_NODES,), jnp.float32)]
        + edge_bufs + edge_bufs + [
            pltpu.VMEM((16,), jnp.float32),
            pltpu.SemaphoreType.DMA,
            pltpu.SemaphoreType.DMA,
            pltpu.SemaphoreType.DMA,
            pltpu.SemaphoreType.DMA,
        ],
    )
    return f(px, py, pz, row, col, bt)


def kernel(positions, edge_index, bond_types, batch):
    px = positions[:, 0]
    py = positions[:, 1]
    pz = positions[:, 2]
    partials, _ = _run(px, py, pz, edge_index[0], edge_index[1], bond_types)
    return jnp.sum(partials) / jnp.float32(N_EDGES)
